# baseline (speedup 1.0000x reference)
import functools

import jax
import jax.numpy as jnp
from jax import lax
from jax.experimental import pallas as pl
from jax.experimental.pallas import tpu as pltpu
from jax.experimental.pallas import tpu_sc as plsc


def _sc_gather128(idx_pad, table128, n_pad):
    info = plsc.get_sparse_core_info()
    NC, NS = info.num_cores, info.num_subcores
    NW = NC * NS
    b_per_w = n_pad // NW

    mesh = plsc.VectorSubcoreMesh(core_axis_name="c", subcore_axis_name="s")

    @functools.partial(
        pl.kernel,
        mesh=mesh,
        out_type=jax.ShapeDtypeStruct((n_pad, 128), jnp.float32),
        scratch_types=[
            pltpu.VMEM((b_per_w,), jnp.int32),
            pltpu.VMEM((b_per_w,), jnp.int32),
            pltpu.VMEM((b_per_w, 128), jnp.float32),
            pltpu.SemaphoreType.DMA,
        ],
    )
    def gather_kernel(idx_hbm, table_hbm, out_hbm, idx_v, row_v, rows_v, sem):
        wid = lax.axis_index("s") * NC + lax.axis_index("c")
        base = wid * b_per_w
        pltpu.sync_copy(idx_hbm.at[pl.ds(base, b_per_w)], idx_v)
        for c in range(b_per_w // 16):
            sl = pl.ds(c * 16, 16)
            row_v[sl] = lax.shift_right_logical(idx_v[sl], 3)
        pltpu.async_copy(table_hbm.at[row_v], rows_v, sem).wait()
        pltpu.sync_copy(rows_v, out_hbm.at[pl.ds(base, b_per_w)])

    return gather_kernel(idx_pad, table128)


def _tc_project(x128, idx2d, posb, W, b2d, n, V, E):
    RB = 32
    grid = n // RB

    def body(x_ref, idx_ref, pos_ref, w_ref, b_ref, o_ref):
        x128 = x_ref[...]
        off = idx_ref[...] & 7
        x16 = jnp.zeros((RB, E), jnp.float32)
        for g in range(8):
            sel = (off == g).astype(jnp.float32)
            x16 = x16 + sel * x128[:, g * E:(g + 1) * E]
        xp = x16 + pos_ref[...]
        o_ref[...] = (
            jnp.dot(xp, w_ref[...], preferred_element_type=jnp.float32)
            + b_ref[...]
        )

    return pl.pallas_call(
        body,
        grid=(grid,),
        in_specs=[
            pl.BlockSpec((RB, 128), lambda i: (i, 0)),
            pl.BlockSpec((RB, 1), lambda i: (i, 0)),
            pl.BlockSpec((RB, E), lambda i: (i, 0)),
            pl.BlockSpec((E, V), lambda i: (0, 0)),
            pl.BlockSpec((1, V), lambda i: (0, 0)),
        ],
        out_specs=pl.BlockSpec((RB, V), lambda i: (i, 0)),
        out_shape=jax.ShapeDtypeStruct((n, V), jnp.float32),
    )(x128, idx2d, posb, W, b2d)


def kernel(idx, tok_table, pos_table, W, b):
    B, T = idx.shape
    V, E = tok_table.shape
    n = B * T

    NW = 32
    chunk = ((n + NW - 1) // NW + 15) // 16 * 16
    n_pad = chunk * NW
    idx_flat = idx.reshape(n).astype(jnp.int32)
    idx_pad = jnp.zeros((n_pad,), jnp.int32).at[:n].set(idx_flat)

    table128 = tok_table.reshape(V // 8, 128)
    x128 = _sc_gather128(idx_pad, table128, n_pad)[:n]

    idx2d = idx_flat.reshape(n, 1)
    posb = jnp.broadcast_to(pos_table[None], (B, T, E)).reshape(n, E)
    out = _tc_project(x128, idx2d, posb, W, b.reshape(1, V), n, V, E)
    return out.reshape(B, T, V)


# speedup vs baseline: 5.0280x; 5.0280x over previous
import functools

import jax
import jax.numpy as jnp
from jax import lax
from jax.experimental import pallas as pl
from jax.experimental.pallas import tpu as pltpu
from jax.experimental.pallas import tpu_sc as plsc

_NBUF = 4


def _sc_gather_cols(idx_pad, tableT, n_pad, E):
    info = plsc.get_sparse_core_info()
    NC, NS = info.num_cores, info.num_subcores
    NW = NC * NS
    b_per_w = n_pad // NW

    mesh = plsc.VectorSubcoreMesh(core_axis_name="c", subcore_axis_name="s")

    @functools.partial(
        pl.kernel,
        mesh=mesh,
        out_type=jax.ShapeDtypeStruct((n_pad, E), jnp.float32),
        scratch_types=[
            pltpu.VMEM(((b_per_w + 15) // 16 * 16,), jnp.int32),
            pltpu.VMEM((_NBUF, E, 128), jnp.float32),
            pltpu.VMEM((b_per_w, E), jnp.float32),
            [pltpu.SemaphoreType.DMA] * _NBUF,
        ],
        compiler_params=pltpu.CompilerParams(needs_layout_passes=False),
    )
    def gather_kernel(idx_hbm, table_hbm, out_hbm, idx_v, panels, xout, sems):
        wid = lax.axis_index("s") * NC + lax.axis_index("c")
        base = wid * b_per_w
        pltpu.sync_copy(idx_hbm.at[pl.ds(base, b_per_w)], idx_v.at[pl.ds(0, b_per_w)])

        krows = lax.iota(jnp.int32, E)
        lane16 = lax.iota(jnp.int32, 16)

        def tok(i):
            chunk = idx_v[pl.ds((i // 16) * 16, 16)]
            return jnp.sum(jnp.where(lane16 == (i % 16), chunk, 0))

        def start(i):
            col0 = (tok(i) >> 7) * 128
            return pltpu.async_copy(
                table_hbm.at[:, pl.ds(col0, 128)],
                panels.at[i % _NBUF],
                sems[i % _NBUF],
            )

        handles = {}
        for i in range(min(_NBUF - 1, b_per_w)):
            handles[i] = start(i)
        for i in range(b_per_w):
            handles.pop(i).wait()
            lane = jnp.broadcast_to(tok(i) & 127, (E,)).astype(jnp.int32)
            col = plsc.load_gather(panels.at[i % _NBUF], [krows, lane])
            xout[i, :] = col
            if i + _NBUF - 1 < b_per_w:
                handles[i + _NBUF - 1] = start(i + _NBUF - 1)

        pltpu.sync_copy(xout, out_hbm.at[pl.ds(base, b_per_w)])

    return gather_kernel(idx_pad, tableT)


def _tc_project(x_pad, posb, W, b2d, n, V, E):
    RB = 32
    grid = n // RB

    def body(x_ref, pos_ref, w_ref, b_ref, o_ref):
        xp = x_ref[...] + pos_ref[...]
        o_ref[...] = (
            jnp.dot(xp, w_ref[...], preferred_element_type=jnp.float32)
            + b_ref[...]
        )

    return pl.pallas_call(
        body,
        grid=(grid,),
        in_specs=[
            pl.BlockSpec((RB, E), lambda i: (i, 0)),
            pl.BlockSpec((RB, E), lambda i: (i, 0)),
            pl.BlockSpec((E, V), lambda i: (0, 0)),
            pl.BlockSpec((1, V), lambda i: (0, 0)),
        ],
        out_specs=pl.BlockSpec((RB, V), lambda i: (i, 0)),
        out_shape=jax.ShapeDtypeStruct((n, V), jnp.float32),
    )(x_pad, posb, W, b2d)


def kernel(idx, tok_table, pos_table, W, b):
    B, T = idx.shape
    V, E = tok_table.shape
    n = B * T

    idx_flat = idx.T.reshape(n).astype(jnp.int32)

    NW = 32
    chunk = ((n + NW - 1) // NW + 7) // 8 * 8
    n_pad = chunk * NW
    idx_pad = jnp.zeros((n_pad,), jnp.int32).at[:n].set(idx_flat)

    tableT = tok_table.T
    x_pad = _sc_gather_cols(idx_pad, tableT, n_pad, E)

    posb = jnp.repeat(pos_table, B, axis=0)
    out = _tc_project(x_pad, posb, W, b.reshape(1, V), n, V, E)
    return out.reshape(T, B, V).transpose(1, 0, 2)
